# baseline (speedup 1.0000x reference)
import functools

import jax
import jax.numpy as jnp
from jax import lax
from jax.experimental import pallas as pl
from jax.experimental.pallas import tpu as pltpu
from jax.experimental.pallas import tpu_sc as plsc

N = 100000
E = 3200000
CHUNK = 128
NC, NS = 2, 16
NW = NC * NS
TB = 16
NT = 50
RPW = TB * NT
EP = NW * RPW * CHUNK
NPAD = EP - E
NP = 102400
NDUMMY = NP - N
SPAN = NP // NS
ZCH = SPAN // 32
NF = NP // 8
GB = 4


def _mesh():
    return plsc.VectorSubcoreMesh(core_axis_name="c", subcore_axis_name="s",
                                  num_cores=NC, num_subcores=NS)


def _params():
    return pltpu.CompilerParams(use_tc_tiling_on_sc=False)


@functools.lru_cache(maxsize=None)
def _make_sc_degree():
    return functools.partial(
        pl.kernel,
        out_type=jax.ShapeDtypeStruct((NC * NP,), jnp.float32),
        mesh=_mesh(),
        compiler_params=_params(),
        scratch_types=[
            pltpu.VMEM((TB, CHUNK), jnp.int32),
            pltpu.VMEM((TB, CHUNK), jnp.int32),
            pltpu.VMEM((CHUNK,), jnp.float32),
            pltpu.VMEM((SPAN,), jnp.float32),
            pltpu.VMEM_SHARED((NP,), jnp.float32),
            pltpu.SemaphoreType.DMA,
            pltpu.SemaphoreType.DMA,
            pltpu.SemaphoreType.DMA,
            pltpu.SemaphoreType.DMA,
        ],
    )(_sc_degree_body)


def _sc_degree_body(dst_hbm, deg_out, idx0, idx1, ones, zeros1, acc,
                    ssem0, ssem1, isem0, isem1):
    c = lax.axis_index("c")
    s = lax.axis_index("s")
    wid = s * NC + c
    base_row = wid * RPW

    for i in range(CHUNK // 16):
        ones[pl.ds(16 * i, 16)] = jnp.ones((16,), jnp.float32)

    @pl.loop(0, SPAN // 16)
    def _(i):
        zeros1[pl.ds(i * 16, 16)] = jnp.zeros((16,), jnp.float32)

    pltpu.sync_copy(zeros1, acc.at[pl.ds(s * SPAN, SPAN)])
    plsc.subcore_barrier()

    idxb = (idx0, idx1)
    ssems = (ssem0, ssem1)
    isems = (isem0, isem1)

    def drain_scatters(p, count):
        for _ in range(count):
            pltpu.make_async_copy(
                deg_out.at[pl.ds(0, CHUNK)], ones, ssems[p]).wait()

    def wait_idx(p):
        pltpu.make_async_copy(
            dst_hbm.at[pl.ds(0, TB)], idxb[p], isems[p]).wait()

    pltpu.sync_copy(dst_hbm.at[pl.ds(base_row, TB)], idxb[0])

    @pl.loop(0, NT, step=2)
    def _(t):
        for dt in range(2):
            tt = t + dt
            p = dt
            if dt == 0:
                @pl.when(t > 0)
                def _():
                    wait_idx(0)
            else:
                wait_idx(1)
            for j in range(TB):
                pltpu.async_copy(ones, acc.at[idxb[p].at[j]], ssems[p],
                                 add=True)
            if dt == 0:
                @pl.when(t > 0)
                def _():
                    drain_scatters(1, TB)
                pltpu.async_copy(
                    dst_hbm.at[pl.ds(base_row + (tt + 1) * TB, TB)],
                    idxb[1], isem1)
            else:
                drain_scatters(0, TB)

                @pl.when(t < NT - 2)
                def _():
                    pltpu.async_copy(
                        dst_hbm.at[pl.ds(base_row + (tt + 1) * TB, TB)],
                        idxb[0], isem0)

    drain_scatters(1, TB)
    plsc.subcore_barrier()

    pltpu.sync_copy(acc.at[pl.ds(s * SPAN, SPAN)], zeros1)
    pltpu.sync_copy(zeros1, deg_out.at[pl.ds(c * NP + s * SPAN, SPAN)])


@functools.lru_cache(maxsize=None)
def _make_sc_agg(F):
    @functools.partial(
        pl.kernel,
        out_type=jax.ShapeDtypeStruct((NC * NP, F), jnp.float32),
        mesh=_mesh(),
        compiler_params=_params(),
        scratch_types=[
            pltpu.VMEM((TB, CHUNK), jnp.int32),
            pltpu.VMEM((TB, CHUNK), jnp.int32),
            pltpu.VMEM((TB, CHUNK), jnp.int32),
            pltpu.VMEM((TB, CHUNK), jnp.int32),
            pltpu.VMEM((GB, CHUNK, F), jnp.float32),
            pltpu.VMEM((GB, CHUNK, F), jnp.float32),
            pltpu.VMEM((ZCH, F), jnp.float32),
            pltpu.VMEM_SHARED((NP, F), jnp.float32),
            pltpu.SemaphoreType.DMA,
            pltpu.SemaphoreType.DMA,
            pltpu.SemaphoreType.DMA,
            pltpu.SemaphoreType.DMA,
            pltpu.SemaphoreType.DMA,
            pltpu.SemaphoreType.DMA,
        ],
    )
    def _sc_agg(src_hbm, dst_hbm, xs_hbm, out_hbm, idxs0, idxs1, idxd0, idxd1,
                rowsA, rowsB, zeros, acc, gsemA, gsemB, ssemA, ssemB,
                isem0, isem1):
        c = lax.axis_index("c")
        s = lax.axis_index("s")
        wid = s * NC + c
        base_row = wid * RPW

        @pl.loop(0, ZCH)
        def _(i):
            zeros[i, :] = jnp.zeros((F,), jnp.float32)

        for k in range(32):
            pltpu.sync_copy(zeros, acc.at[pl.ds(s * SPAN + k * ZCH, ZCH)])

        plsc.subcore_barrier()

        idxs = (idxs0, idxs1)
        idxd = (idxd0, idxd1)
        rows = (rowsA, rowsB)
        gsems = (gsemA, gsemB)
        ssems = (ssemA, ssemB)
        isems = (isem0, isem1)

        def wait_idx(p):
            pltpu.make_async_copy(
                src_hbm.at[pl.ds(0, TB)], idxs[p], isems[p]).wait()
            pltpu.make_async_copy(
                src_hbm.at[pl.ds(0, TB)], idxd[p], isems[p]).wait()

        def load_idx(tt, p, sem):
            pltpu.async_copy(
                src_hbm.at[pl.ds(base_row + tt * TB, TB)], idxs[p], sem)
            pltpu.async_copy(
                dst_hbm.at[pl.ds(base_row + tt * TB, TB)], idxd[p], sem)

        def drain_scatters(b, count):
            for _ in range(count):
                pltpu.make_async_copy(
                    xs_hbm.at[pl.ds(0, CHUNK)], rows[b].at[0], ssems[b]).wait()

        pltpu.sync_copy(src_hbm.at[pl.ds(base_row, TB)], idxs[0])
        pltpu.sync_copy(dst_hbm.at[pl.ds(base_row, TB)], idxd[0])

        @pl.loop(0, NT, step=2)
        def _(t):
            for dt in range(2):
                tt = t + dt
                p = dt
                if dt == 0:
                    @pl.when(t > 0)
                    def _():
                        wait_idx(0)
                else:
                    wait_idx(1)
                gd = [None, None]

                def pre_and_gather(g, guard_first):
                    b = g % 2
                    if guard_first and dt == 0:
                        @pl.when(t > 0)
                        def _():
                            drain_scatters(b, GB)
                    else:
                        drain_scatters(b, GB)
                    gd[b] = [pltpu.async_copy(
                        xs_hbm.at[idxs[p].at[g * GB + j]], rows[b].at[j],
                        gsems[b]) for j in range(GB)]

                def finish(g):
                    b = g % 2
                    for j in range(GB):
                        gd[b][j].wait()
                        pltpu.async_copy(rows[b].at[j],
                                         acc.at[idxd[p].at[g * GB + j]],
                                         ssems[b], add=True)

                pre_and_gather(0, True)
                pre_and_gather(1, True)
                if dt == 0:
                    load_idx(tt + 1, 1, isem1)
                else:
                    @pl.when(t < NT - 2)
                    def _():
                        load_idx(tt + 1, 0, isem0)
                finish(0)
                pre_and_gather(2, False)
                finish(1)
                pre_and_gather(3, False)
                finish(2)
                finish(3)

        drain_scatters(0, GB)
        drain_scatters(1, GB)
        plsc.subcore_barrier()

        for k in range(32):
            pltpu.sync_copy(acc.at[pl.ds(s * SPAN + k * ZCH, ZCH)], zeros)
            pltpu.sync_copy(
                zeros, out_hbm.at[pl.ds(c * NP + s * SPAN + k * ZCH, ZCH)])

    return _sc_agg


_RB = 1600
_GF = NF // _RB


def _tc_pre1_body(degf_ref, xf_ref, bx_ref, dinv_ref, xs1_ref):
    d = degf_ref[0] + degf_ref[1] + 1.0
    dinv = lax.rsqrt(d)
    xwf = jnp.dot(xf_ref[...], bx_ref[...], preferred_element_type=jnp.float32)
    dinv_ref[...] = dinv
    xs1_ref[...] = dinv * xwf


def _tc_pre1(degf, xf, Bx):
    return pl.pallas_call(
        _tc_pre1_body,
        grid=(_GF,),
        in_specs=[
            pl.BlockSpec((NC, _RB, 128), lambda i: (0, i, 0)),
            pl.BlockSpec((_RB, 24), lambda i: (i, 0)),
            pl.BlockSpec((24, 128), lambda i: (0, 0)),
        ],
        out_specs=[
            pl.BlockSpec((_RB, 128), lambda i: (i, 0)),
            pl.BlockSpec((_RB, 128), lambda i: (i, 0)),
        ],
        out_shape=[
            jax.ShapeDtypeStruct((NF, 128), jnp.float32),
            jax.ShapeDtypeStruct((NF, 128), jnp.float32),
        ],
    )(degf, xf, Bx)


def _tc_mid_body(aggf_ref, xs1_ref, dinv_ref, b1_ref, xs2_ref):
    t = aggf_ref[0] + aggf_ref[1] + xs1_ref[...]
    dinv = dinv_ref[...]
    h = jnp.maximum(dinv * t + b1_ref[...], 0.0)
    xs2_ref[...] = dinv * h


def _tc_mid(agg1f, xs1f, dinvf, b1t):
    return pl.pallas_call(
        _tc_mid_body,
        grid=(_GF,),
        in_specs=[
            pl.BlockSpec((NC, _RB, 128), lambda i: (0, i, 0)),
            pl.BlockSpec((_RB, 128), lambda i: (i, 0)),
            pl.BlockSpec((_RB, 128), lambda i: (i, 0)),
            pl.BlockSpec((1, 128), lambda i: (0, 0)),
        ],
        out_specs=pl.BlockSpec((_RB, 128), lambda i: (i, 0)),
        out_shape=jax.ShapeDtypeStruct((NF, 128), jnp.float32),
    )(agg1f, xs1f, dinvf, b1t)


def _tc_post_body(aggf_ref, xs2_ref, dinv_ref, b2m_ref, b2t_ref, out_ref):
    t = dinv_ref[...] * (aggf_ref[0] + aggf_ref[1] + xs2_ref[...])
    out_ref[...] = (
        jnp.dot(t, b2m_ref[...], preferred_element_type=jnp.float32)
        + b2t_ref[...])


def _tc_post(agg2f, xs2f, dinvf, B2, b2t):
    return pl.pallas_call(
        _tc_post_body,
        grid=(_GF,),
        in_specs=[
            pl.BlockSpec((NC, _RB, 128), lambda i: (0, i, 0)),
            pl.BlockSpec((_RB, 128), lambda i: (i, 0)),
            pl.BlockSpec((_RB, 128), lambda i: (i, 0)),
            pl.BlockSpec((128, 128), lambda i: (0, 0)),
            pl.BlockSpec((1, 128), lambda i: (0, 0)),
        ],
        out_specs=pl.BlockSpec((_RB, 128), lambda i: (i, 0)),
        out_shape=jax.ShapeDtypeStruct((NF, 128), jnp.float32),
    )(agg2f, xs2f, dinvf, B2, b2t)


def kernel(x, edge_index, W1, b1, W2, b2):
    src = edge_index[0].astype(jnp.int32)
    dst = edge_index[1].astype(jnp.int32)
    pad_src = jnp.arange(NPAD, dtype=jnp.int32) % N
    pad_dst = N + (jnp.arange(NPAD, dtype=jnp.int32) % NDUMMY)
    src = jnp.concatenate([src, pad_src]).reshape(-1, CHUNK)
    dst = jnp.concatenate([dst, pad_dst]).reshape(-1, CHUNK)

    xf = jnp.pad(x, ((0, NP - N), (0, 0))).reshape(NF, 24)
    Bx = jnp.kron(jnp.eye(8, dtype=jnp.float32), W1)
    W2p = jnp.pad(W2, ((0, 0), (0, 9)))
    B2 = jnp.kron(jnp.eye(8, dtype=jnp.float32), W2p)
    b1t = jnp.tile(b1, 8).reshape(1, 128)
    b2t = jnp.tile(jnp.pad(b2, (0, 9)), 8).reshape(1, 128)

    deg_p = _make_sc_degree()(dst)
    degf = jnp.broadcast_to(deg_p.reshape(NC, NF, 8, 1),
                            (NC, NF, 8, 16)).reshape(NC, NF, 128)
    dinvf, xs1f = _tc_pre1(degf, xf, Bx)
    agg = _make_sc_agg(16)
    agg1f = agg(src, dst, xs1f.reshape(NP, 16)).reshape(NC, NF, 128)
    xs2f = _tc_mid(agg1f, xs1f, dinvf, b1t)
    agg2f = agg(src, dst, xs2f.reshape(NP, 16)).reshape(NC, NF, 128)
    outf = _tc_post(agg2f, xs2f, dinvf, B2, b2t)
    return outf.reshape(NP, 16)[:N, :7]


# speedup vs baseline: 1.0411x; 1.0411x over previous
import functools

import jax
import jax.numpy as jnp
from jax import lax
from jax.experimental import pallas as pl
from jax.experimental.pallas import tpu as pltpu
from jax.experimental.pallas import tpu_sc as plsc

N = 100000
E = 3200000
CHUNK = 128
NC, NS = 2, 16
NW = NC * NS
TB = 16
NT = 50
RPW = TB * NT
EP = NW * RPW * CHUNK
NPAD = EP - E
NP = 102400
NDUMMY = NP - N
SPAN = NP // NS
ZCH = SPAN // 32
NF = NP // 8
GB = 4


def _mesh():
    return plsc.VectorSubcoreMesh(core_axis_name="c", subcore_axis_name="s",
                                  num_cores=NC, num_subcores=NS)


def _params():
    return pltpu.CompilerParams(use_tc_tiling_on_sc=False)


@functools.lru_cache(maxsize=None)
def _make_sc_degree():
    return functools.partial(
        pl.kernel,
        out_type=jax.ShapeDtypeStruct((NC * NP,), jnp.float32),
        mesh=_mesh(),
        compiler_params=_params(),
        scratch_types=[
            pltpu.VMEM((TB, CHUNK), jnp.int32),
            pltpu.VMEM((TB, CHUNK), jnp.int32),
            pltpu.VMEM((CHUNK,), jnp.float32),
            pltpu.VMEM((SPAN,), jnp.float32),
            pltpu.VMEM_SHARED((NP,), jnp.float32),
            pltpu.SemaphoreType.DMA,
            pltpu.SemaphoreType.DMA,
            pltpu.SemaphoreType.DMA,
            pltpu.SemaphoreType.DMA,
        ],
    )(_sc_degree_body)


def _sc_degree_body(dst_hbm, deg_out, idx0, idx1, ones, zeros1, acc,
                    ssem0, ssem1, isem0, isem1):
    c = lax.axis_index("c")
    s = lax.axis_index("s")
    wid = s * NC + c
    base_row = wid * RPW

    for i in range(CHUNK // 16):
        ones[pl.ds(16 * i, 16)] = jnp.ones((16,), jnp.float32)

    @pl.loop(0, SPAN // 16)
    def _(i):
        zeros1[pl.ds(i * 16, 16)] = jnp.zeros((16,), jnp.float32)

    pltpu.sync_copy(zeros1, acc.at[pl.ds(s * SPAN, SPAN)])
    plsc.subcore_barrier()

    idxb = (idx0, idx1)
    ssems = (ssem0, ssem1)
    isems = (isem0, isem1)

    def drain_scatters(p, count):
        for _ in range(count):
            pltpu.make_async_copy(
                deg_out.at[pl.ds(0, CHUNK)], ones, ssems[p]).wait()

    def wait_idx(p):
        pltpu.make_async_copy(
            dst_hbm.at[pl.ds(0, TB)], idxb[p], isems[p]).wait()

    pltpu.sync_copy(dst_hbm.at[pl.ds(base_row, TB)], idxb[0])

    @pl.loop(0, NT, step=2)
    def _(t):
        for dt in range(2):
            tt = t + dt
            p = dt
            if dt == 0:
                @pl.when(t > 0)
                def _():
                    wait_idx(0)
            else:
                wait_idx(1)
            for j in range(TB):
                pltpu.async_copy(ones, acc.at[idxb[p].at[j]], ssems[p],
                                 add=True)
            if dt == 0:
                @pl.when(t > 0)
                def _():
                    drain_scatters(1, TB)
                pltpu.async_copy(
                    dst_hbm.at[pl.ds(base_row + (tt + 1) * TB, TB)],
                    idxb[1], isem1)
            else:
                drain_scatters(0, TB)

                @pl.when(t < NT - 2)
                def _():
                    pltpu.async_copy(
                        dst_hbm.at[pl.ds(base_row + (tt + 1) * TB, TB)],
                        idxb[0], isem0)

    drain_scatters(1, TB)
    plsc.subcore_barrier()

    pltpu.sync_copy(acc.at[pl.ds(s * SPAN, SPAN)], zeros1)
    pltpu.sync_copy(zeros1, deg_out.at[pl.ds(c * NP + s * SPAN, SPAN)])


@functools.lru_cache(maxsize=None)
def _make_sc_agg(F):
    @functools.partial(
        pl.kernel,
        out_type=jax.ShapeDtypeStruct((NC * NP, F), jnp.float32),
        mesh=_mesh(),
        compiler_params=_params(),
        scratch_types=[
            pltpu.VMEM((TB, CHUNK), jnp.int32),
            pltpu.VMEM((TB, CHUNK), jnp.int32),
            pltpu.VMEM((TB, CHUNK), jnp.int32),
            pltpu.VMEM((TB, CHUNK), jnp.int32),
            pltpu.VMEM((GB, CHUNK, F), jnp.float32),
            pltpu.VMEM((GB, CHUNK, F), jnp.float32),
            pltpu.VMEM((ZCH, F), jnp.float32),
            pltpu.VMEM_SHARED((NP, F), jnp.float32),
            pltpu.SemaphoreType.DMA,
            pltpu.SemaphoreType.DMA,
            pltpu.SemaphoreType.DMA,
            pltpu.SemaphoreType.DMA,
            pltpu.SemaphoreType.DMA,
            pltpu.SemaphoreType.DMA,
        ],
    )
    def _sc_agg(src_hbm, dst_hbm, xs_hbm, out_hbm, idxs0, idxs1, idxd0, idxd1,
                rowsA, rowsB, zeros, acc, gsemA, gsemB, ssemA, ssemB,
                isem0, isem1):
        c = lax.axis_index("c")
        s = lax.axis_index("s")
        wid = s * NC + c
        base_row = wid * RPW

        @pl.loop(0, ZCH)
        def _(i):
            zeros[i, :] = jnp.zeros((F,), jnp.float32)

        for k in range(32):
            pltpu.sync_copy(zeros, acc.at[pl.ds(s * SPAN + k * ZCH, ZCH)])

        plsc.subcore_barrier()

        idxs = (idxs0, idxs1)
        idxd = (idxd0, idxd1)
        rows = (rowsA, rowsB)
        gsems = (gsemA, gsemB)
        ssems = (ssemA, ssemB)
        isems = (isem0, isem1)

        def wait_idx(p):
            pltpu.make_async_copy(
                src_hbm.at[pl.ds(0, TB)], idxs[p], isems[p]).wait()
            pltpu.make_async_copy(
                src_hbm.at[pl.ds(0, TB)], idxd[p], isems[p]).wait()

        def load_idx(tt, p, sem):
            pltpu.async_copy(
                src_hbm.at[pl.ds(base_row + tt * TB, TB)], idxs[p], sem)
            pltpu.async_copy(
                dst_hbm.at[pl.ds(base_row + tt * TB, TB)], idxd[p], sem)

        def drain_scatters(b, count):
            for _ in range(count):
                pltpu.make_async_copy(
                    xs_hbm.at[pl.ds(0, CHUNK)], rows[b].at[0], ssems[b]).wait()

        pltpu.sync_copy(src_hbm.at[pl.ds(base_row, TB)], idxs[0])
        pltpu.sync_copy(dst_hbm.at[pl.ds(base_row, TB)], idxd[0])

        @pl.loop(0, NT, step=2)
        def _(t):
            for dt in range(2):
                tt = t + dt
                p = dt
                if dt == 0:
                    @pl.when(t > 0)
                    def _():
                        wait_idx(0)
                else:
                    wait_idx(1)
                gd = [None, None]

                def pre_and_gather(g, guard_first):
                    b = g % 2
                    if guard_first and dt == 0:
                        @pl.when(t > 0)
                        def _():
                            drain_scatters(b, GB)
                    else:
                        drain_scatters(b, GB)
                    gd[b] = [pltpu.async_copy(
                        xs_hbm.at[idxs[p].at[g * GB + j]], rows[b].at[j],
                        gsems[b]) for j in range(GB)]

                def finish(g):
                    b = g % 2
                    for j in range(GB):
                        gd[b][j].wait()
                        pltpu.async_copy(rows[b].at[j],
                                         acc.at[idxd[p].at[g * GB + j]],
                                         ssems[b], add=True)

                pre_and_gather(0, True)
                pre_and_gather(1, True)
                if dt == 0:
                    load_idx(tt + 1, 1, isem1)
                else:
                    @pl.when(t < NT - 2)
                    def _():
                        load_idx(tt + 1, 0, isem0)
                finish(0)
                pre_and_gather(2, False)
                finish(1)
                pre_and_gather(3, False)
                finish(2)
                finish(3)

        drain_scatters(0, GB)
        drain_scatters(1, GB)
        plsc.subcore_barrier()

        for k in range(32):
            pltpu.sync_copy(acc.at[pl.ds(s * SPAN + k * ZCH, ZCH)], zeros)
            pltpu.sync_copy(
                zeros, out_hbm.at[pl.ds(c * NP + s * SPAN + k * ZCH, ZCH)])

    return _sc_agg


_RB = 1600
_GF = NF // _RB


def _tc_pre1_body(degf_ref, xf_ref, bx_ref, dinv_ref, xs1_ref):
    d = degf_ref[0] + degf_ref[1] + 1.0
    dinv = lax.rsqrt(d)
    xwf = jnp.dot(xf_ref[...], bx_ref[...], preferred_element_type=jnp.float32)
    dinv_ref[...] = dinv
    xs1_ref[...] = dinv * xwf


def _tc_pre1(degf, xf, Bx):
    return pl.pallas_call(
        _tc_pre1_body,
        grid=(_GF,),
        in_specs=[
            pl.BlockSpec((NC, _RB, 128), lambda i: (0, i, 0)),
            pl.BlockSpec((_RB, 24), lambda i: (i, 0)),
            pl.BlockSpec((24, 128), lambda i: (0, 0)),
        ],
        out_specs=[
            pl.BlockSpec((_RB, 128), lambda i: (i, 0)),
            pl.BlockSpec((_RB, 128), lambda i: (i, 0)),
        ],
        out_shape=[
            jax.ShapeDtypeStruct((NF, 128), jnp.float32),
            jax.ShapeDtypeStruct((NF, 128), jnp.float32),
        ],
    )(degf, xf, Bx)


def _tc_mid_body(aggf_ref, xs1_ref, dinv_ref, b1_ref, xs2_ref):
    t = aggf_ref[0] + aggf_ref[1] + xs1_ref[...]
    dinv = dinv_ref[...]
    h = jnp.maximum(dinv * t + b1_ref[...], 0.0)
    xs2_ref[...] = dinv * h


def _tc_mid(agg1f, xs1f, dinvf, b1t):
    return pl.pallas_call(
        _tc_mid_body,
        grid=(_GF,),
        in_specs=[
            pl.BlockSpec((NC, _RB, 128), lambda i: (0, i, 0)),
            pl.BlockSpec((_RB, 128), lambda i: (i, 0)),
            pl.BlockSpec((_RB, 128), lambda i: (i, 0)),
            pl.BlockSpec((1, 128), lambda i: (0, 0)),
        ],
        out_specs=pl.BlockSpec((_RB, 128), lambda i: (i, 0)),
        out_shape=jax.ShapeDtypeStruct((NF, 128), jnp.float32),
    )(agg1f, xs1f, dinvf, b1t)


def _tc_post_body(aggf_ref, xs2_ref, dinv_ref, b2m_ref, b2t_ref, out_ref):
    t = dinv_ref[...] * (aggf_ref[0] + aggf_ref[1] + xs2_ref[...])
    out_ref[...] = (
        jnp.dot(t, b2m_ref[...], preferred_element_type=jnp.float32)
        + b2t_ref[...])


def _tc_post(agg2f, xs2f, dinvf, B2, b2t):
    return pl.pallas_call(
        _tc_post_body,
        grid=(_GF,),
        in_specs=[
            pl.BlockSpec((NC, _RB, 128), lambda i: (0, i, 0)),
            pl.BlockSpec((_RB, 128), lambda i: (i, 0)),
            pl.BlockSpec((_RB, 128), lambda i: (i, 0)),
            pl.BlockSpec((128, 128), lambda i: (0, 0)),
            pl.BlockSpec((1, 128), lambda i: (0, 0)),
        ],
        out_specs=pl.BlockSpec((_RB, 128), lambda i: (i, 0)),
        out_shape=jax.ShapeDtypeStruct((NF, 128), jnp.float32),
    )(agg2f, xs2f, dinvf, B2, b2t)


def kernel(x, edge_index, W1, b1, W2, b2):
    src = edge_index[0].astype(jnp.int32)
    dst = edge_index[1].astype(jnp.int32)
    pad_src = jnp.arange(NPAD, dtype=jnp.int32) % N
    pad_dst = N + (jnp.arange(NPAD, dtype=jnp.int32) % NDUMMY)
    src = jnp.concatenate([src, pad_src]).reshape(-1, CHUNK)
    dst = jnp.concatenate([dst, pad_dst]).reshape(-1, CHUNK)

    xf = jnp.pad(x, ((0, NP - N), (0, 0))).reshape(NF, 24)
    Bx = jnp.kron(jnp.eye(8, dtype=jnp.float32), W1)
    W2p = jnp.pad(W2, ((0, 0), (0, 9)))
    B2 = jnp.kron(jnp.eye(8, dtype=jnp.float32), W2p)
    b1t = jnp.tile(b1, 8).reshape(1, 128)
    b2t = jnp.tile(jnp.pad(b2, (0, 9)), 8).reshape(1, 128)

    deg_p = _make_sc_degree()(dst)
    degf = jnp.broadcast_to(deg_p.reshape(NC, NF, 8, 1),
                            (NC, NF, 8, 16)).reshape(NC, NF, 128)
    dinvf, xs1f = _tc_pre1(degf, xf, Bx)
    agg = _make_sc_agg(16)
    agg1f = agg(src, dst, xs1f.reshape(NP, 16)).reshape(NC, NF, 128)
    xs2f = _tc_mid(agg1f, xs1f, dinvf, b1t)
    agg2f = agg(src, dst, xs2f.reshape(NP, 16)).reshape(NC, NF, 128)
    outf = _tc_post(agg2f, xs2f, dinvf, B2, b2t)
    return outf[:N // 8].reshape(N, 16)[:, :7]
